# Initial kernel scaffold; baseline (speedup 1.0000x reference)
import jax
import jax.numpy as jnp
from jax.experimental import pallas as pl


def kernel(x, W1, g1, b1, W2, g2, b2, W3, g3, b3, W4, g4, b4, W5, g5, b5, L1, gl1, bl1, L2, gl2, bl2, L3, bL3):
    raise NotImplementedError("write your pallas kernel here")


# speedup vs baseline: 9.3836x; 9.3836x over previous
import functools

import jax
import jax.numpy as jnp
from jax import lax
from jax.experimental import pallas as pl
from jax.experimental.pallas import tpu as pltpu
from jax.experimental.pallas import tpu_sc as plsc

KNN = 20
B = 16
N = 2048
M = B * N
CP = 128
NEG = -3.0e38

SC_CORES = 2
SC_SUBCORES = 16
SC_WORKERS = SC_CORES * SC_SUBCORES
SC_LANES = 16


def _dist_topk_body(xr_ref, xa_ref, idx_ref):
    b = pl.program_id(0)
    xr = xr_ref[0]
    xa = xa_ref[0]
    inner = lax.dot_general(xr.astype(jnp.bfloat16), xa.astype(jnp.bfloat16),
                            (((1,), (1,)), ((), ())),
                            preferred_element_type=jnp.float32)
    xxr = jnp.sum(xr * xr, axis=1, keepdims=True)
    xxa = jnp.sum(xa * xa, axis=1)[None, :]
    pd = 2.0 * inner - xxr - xxa
    rt, n = pd.shape
    iota = lax.broadcasted_iota(jnp.int32, (rt, n), 1)
    cols = []
    for _ in range(KNN):
        m = jnp.max(pd, axis=1, keepdims=True)
        cand = jnp.where(pd == m, iota, n)
        am = jnp.min(cand, axis=1, keepdims=True)
        cols.append(am)
        pd = jnp.where(iota == am, NEG, pd)
    idx_ref[0] = jnp.concatenate(cols, axis=1) + b * N


def _dist_topk(xp, rt=256):
    c = xp.shape[-1]
    return pl.pallas_call(
        _dist_topk_body,
        grid=(B, N // rt),
        in_specs=[
            pl.BlockSpec((1, rt, c), lambda b, i: (b, i, 0)),
            pl.BlockSpec((1, N, c), lambda b, i: (b, 0, 0)),
        ],
        out_specs=pl.BlockSpec((1, rt, KNN), lambda b, i: (b, i, 0)),
        out_shape=jax.ShapeDtypeStruct((B, N, KNN), jnp.int32),
    )(xp, xp)


_SC_P = 4


def _sc_gather_scatter(idx_flat, table, pattern):
    p = _SC_P
    per_w = M // SC_WORKERS
    ngroups = per_w // p
    nvec = (p * KNN) // SC_LANES
    mesh = plsc.VectorSubcoreMesh(
        core_axis_name="c", subcore_axis_name="s",
        num_cores=SC_CORES, num_subcores=SC_SUBCORES)

    @functools.partial(
        pl.kernel,
        mesh=mesh,
        out_type=jax.ShapeDtypeStruct((KNN * M, CP), jnp.float32),
        scratch_types=[
            pltpu.VMEM((p * KNN,), jnp.int32),
            pltpu.VMEM((p * KNN,), jnp.int32),
            pltpu.VMEM((p * KNN,), jnp.int32),
            pltpu.VMEM((p * KNN, CP), jnp.float32),
            pltpu.SemaphoreType.DMA,
            pltpu.SemaphoreType.DMA,
        ],
    )
    def k(idx_hbm, tab_hbm, pat_hbm, g_hbm,
          idx_v, pat_v, didx_v, rows_v, sem_g, sem_s):
        wid = lax.axis_index("s") * SC_CORES + lax.axis_index("c")
        base = wid * per_w
        pltpu.sync_copy(pat_hbm, pat_v)

        def group(g, _):
            pt0 = base + g * p
            pltpu.sync_copy(idx_hbm.at[pl.ds(pt0 * KNN, p * KNN)], idx_v)
            for v in range(nvec):
                sl = pl.ds(v * SC_LANES, SC_LANES)
                didx_v[sl] = pat_v[sl] + pt0
            pltpu.async_copy(tab_hbm.at[idx_v], rows_v, sem_g).wait()
            pltpu.async_copy(rows_v, g_hbm.at[didx_v], sem_s).wait()
            return 0

        lax.fori_loop(0, ngroups, group, 0)

    return k(idx_flat, table, pattern)


def _econv_body(x_ref, g_ref, wf_ref, ymax_ref, mean_ref, sqv_ref, acc_ref):
    i = pl.program_id(0)

    @pl.when(i == 0)
    def _():
        acc_ref[...] = jnp.zeros_like(acc_ref)

    x = x_ref[...]
    wf = wf_ref[...].astype(jnp.bfloat16)
    c0 = wf.shape[0] // 2
    o = wf.shape[1]
    pt = x.shape[0]
    amax = jnp.full((pt, o), NEG, jnp.float32)
    asum = jnp.zeros((pt, o), jnp.float32)
    asq = jnp.zeros((pt, o), jnp.float32)
    for kk in range(KNN):
        ec = jnp.concatenate([(g_ref[kk] - x)[:, :c0], x[:, :c0]],
                             axis=1).astype(jnp.bfloat16)
        yk = lax.dot_general(ec, wf, (((1,), (0,)), ((), ())),
                             preferred_element_type=jnp.float32)
        amax = jnp.maximum(amax, yk)
        asum = asum + yk
        asq = asq + yk * yk
    ymax_ref[...] = amax
    acc_ref[...] += jnp.concatenate([
        jnp.sum(asum, 0, keepdims=True),
        jnp.sum(asq, 0, keepdims=True),
        jnp.zeros((6, o), jnp.float32),
    ], axis=0)

    @pl.when(i == pl.num_programs(0) - 1)
    def _():
        a = acc_ref[...]
        cnt = float(M * KNN)
        mean = a[0:1] / cnt
        var = a[1:2] / cnt - mean * mean
        mean_ref[...] = mean
        sqv_ref[...] = jnp.sqrt(var + 1e-5)


def _econv(x, g, wfull, o, pt=256):
    c2 = wfull.shape[0]
    return pl.pallas_call(
        _econv_body,
        grid=(M // pt,),
        in_specs=[
            pl.BlockSpec((pt, CP), lambda i: (i, 0)),
            pl.BlockSpec((KNN, pt, CP), lambda i: (0, i, 0)),
            pl.BlockSpec((c2, o), lambda i: (0, 0)),
        ],
        out_specs=[
            pl.BlockSpec((pt, o), lambda i: (i, 0)),
            pl.BlockSpec((1, o), lambda i: (0, 0)),
            pl.BlockSpec((1, o), lambda i: (0, 0)),
        ],
        out_shape=[
            jax.ShapeDtypeStruct((M, o), jnp.float32),
            jax.ShapeDtypeStruct((1, o), jnp.float32),
            jax.ShapeDtypeStruct((1, o), jnp.float32),
        ],
        scratch_shapes=[pltpu.VMEM((8, o), jnp.float32)],
    )(x, g.reshape(KNN, M, CP), wfull)


def _combine_body(ymax_ref, mean_ref, sqv_ref, g_ref, b_ref, out_ref):
    y = ymax_ref[...]
    y = (y - mean_ref[...]) / sqv_ref[...] * g_ref[...] + b_ref[...]
    y = jnp.where(y >= 0, y, 0.2 * y)
    o = y.shape[1]
    pad = out_ref.shape[1] - o
    if pad:
        y = jnp.concatenate([y, jnp.zeros((y.shape[0], pad), jnp.float32)], 1)
    out_ref[...] = y


def _combine(ymax, mean, sqv, g, b, o, opad, rt=1024):
    return pl.pallas_call(
        _combine_body,
        grid=(M // rt,),
        in_specs=[
            pl.BlockSpec((rt, o), lambda i: (i, 0)),
            pl.BlockSpec((1, o), lambda i: (0, 0)),
            pl.BlockSpec((1, o), lambda i: (0, 0)),
            pl.BlockSpec((1, o), lambda i: (0, 0)),
            pl.BlockSpec((1, o), lambda i: (0, 0)),
        ],
        out_specs=pl.BlockSpec((rt, opad), lambda i: (i, 0)),
        out_shape=jax.ShapeDtypeStruct((M, opad), jnp.float32),
    )(ymax, mean, sqv, g, b)


def _edge_layer(xpad, c0, w, g, b, opad, idx_pattern):
    o = w.shape[0]
    wfull = w.T

    idx = _dist_topk(xpad.reshape(B, N, CP))
    gth = _sc_gather_scatter(idx.reshape(M * KNN), xpad, idx_pattern)
    ymax, mean, sqv = _econv(xpad, gth, wfull, o)
    return _combine(ymax, mean, sqv, g.reshape(1, o), b.reshape(1, o),
                    o, opad)


def _conv5_p1_body(x_ref, w_ref, y_ref, mean_ref, sqv_ref, acc_ref):
    i = pl.program_id(0)

    @pl.when(i == 0)
    def _():
        acc_ref[...] = jnp.zeros_like(acc_ref)

    y = lax.dot_general(x_ref[...].astype(jnp.bfloat16),
                        w_ref[...].astype(jnp.bfloat16),
                        (((1,), (0,)), ((), ())),
                        preferred_element_type=jnp.float32)
    y_ref[...] = y
    o = y.shape[1]
    acc_ref[...] += jnp.concatenate([
        jnp.sum(y, 0, keepdims=True),
        jnp.sum(y * y, 0, keepdims=True),
        jnp.zeros((6, o), jnp.float32),
    ], axis=0)

    @pl.when(i == pl.num_programs(0) - 1)
    def _():
        a = acc_ref[...]
        mean = a[0:1] / float(M)
        var = a[1:2] / float(M) - mean * mean
        mean_ref[...] = mean
        sqv_ref[...] = jnp.sqrt(var + 1e-5)


def _conv5_p1(xcat, w5t, rt=512):
    emb = w5t.shape[1]
    return pl.pallas_call(
        _conv5_p1_body,
        grid=(M // rt,),
        in_specs=[
            pl.BlockSpec((rt, 512), lambda i: (i, 0)),
            pl.BlockSpec((512, emb), lambda i: (0, 0)),
        ],
        out_specs=[
            pl.BlockSpec((rt, emb), lambda i: (i, 0)),
            pl.BlockSpec((1, emb), lambda i: (0, 0)),
            pl.BlockSpec((1, emb), lambda i: (0, 0)),
        ],
        out_shape=[
            jax.ShapeDtypeStruct((M, emb), jnp.float32),
            jax.ShapeDtypeStruct((1, emb), jnp.float32),
            jax.ShapeDtypeStruct((1, emb), jnp.float32),
        ],
        scratch_shapes=[pltpu.VMEM((8, emb), jnp.float32)],
    )(xcat, w5t)


def _conv5_p2_body(y_ref, mean_ref, sqv_ref, g_ref, b_ref,
                   hmax_ref, hmean_ref, accm_ref, accs_ref):
    j = pl.program_id(1)

    @pl.when(j == 0)
    def _():
        accm_ref[...] = jnp.full_like(accm_ref, NEG)
        accs_ref[...] = jnp.zeros_like(accs_ref)

    z = y_ref[0]
    z = (z - mean_ref[...]) / sqv_ref[...] * g_ref[...] + b_ref[...]
    z = jnp.where(z >= 0, z, 0.2 * z)
    accm_ref[0:1] = jnp.maximum(accm_ref[0:1], jnp.max(z, 0, keepdims=True))
    accs_ref[0:1] += jnp.sum(z, 0, keepdims=True)

    @pl.when(j == pl.num_programs(1) - 1)
    def _():
        hmax_ref[0] = accm_ref[0:1]
        hmean_ref[0] = accs_ref[0:1] / float(N)


def _conv5_p2(y5, mean, sqv, g, b, rt=512):
    emb = y5.shape[2]
    return pl.pallas_call(
        _conv5_p2_body,
        grid=(B, N // rt),
        in_specs=[
            pl.BlockSpec((1, rt, emb), lambda bb, j: (bb, j, 0)),
            pl.BlockSpec((1, emb), lambda bb, j: (0, 0)),
            pl.BlockSpec((1, emb), lambda bb, j: (0, 0)),
            pl.BlockSpec((1, emb), lambda bb, j: (0, 0)),
            pl.BlockSpec((1, emb), lambda bb, j: (0, 0)),
        ],
        out_specs=[
            pl.BlockSpec((1, 1, emb), lambda bb, j: (bb, 0, 0)),
            pl.BlockSpec((1, 1, emb), lambda bb, j: (bb, 0, 0)),
        ],
        out_shape=[
            jax.ShapeDtypeStruct((B, 1, emb), jnp.float32),
            jax.ShapeDtypeStruct((B, 1, emb), jnp.float32),
        ],
        scratch_shapes=[
            pltpu.VMEM((8, emb), jnp.float32),
            pltpu.VMEM((8, emb), jnp.float32),
        ],
    )(y5, mean, sqv, g, b)


def _head_body(h_ref, l1_ref, g1_ref, b1_ref, l2_ref, g2_ref, b2_ref,
               l3_ref, b3_ref, out_ref):
    def lin(t, w_ref):
        return lax.dot_general(t.astype(jnp.bfloat16),
                               w_ref[...].astype(jnp.bfloat16),
                               (((1,), (0,)), ((), ())),
                               preferred_element_type=jnp.float32)

    def lin_bn(t, w_ref, g_ref, b_ref):
        y = lin(t, w_ref)
        m = jnp.mean(y, 0, keepdims=True)
        v = jnp.mean((y - m) * (y - m), 0, keepdims=True)
        y = (y - m) / jnp.sqrt(v + 1e-5) * g_ref[...] + b_ref[...]
        return jnp.where(y >= 0, y, 0.2 * y)

    t = lin_bn(h_ref[...], l1_ref, g1_ref, b1_ref)
    t = lin_bn(t, l2_ref, g2_ref, b2_ref)
    out_ref[...] = lin(t, l3_ref) + b3_ref[...]


def _head(h, l1t, gl1, bl1, l2t, gl2, bl2, l3t, bl3):
    return pl.pallas_call(
        _head_body,
        out_shape=jax.ShapeDtypeStruct((B, l3t.shape[1]), jnp.float32),
    )(h, l1t, gl1.reshape(1, -1), bl1.reshape(1, -1),
      l2t, gl2.reshape(1, -1), bl2.reshape(1, -1), l3t, bl3.reshape(1, -1))


def kernel(x, W1, g1, b1, W2, g2, b2, W3, g3, b3, W4, g4, b4, W5, g5, b5,
           L1, gl1, bl1, L2, gl2, bl2, L3, bL3):
    pat = (jnp.arange(_SC_P * KNN, dtype=jnp.int32) % KNN) * M \
        + (jnp.arange(_SC_P * KNN, dtype=jnp.int32) // KNN)
    xpad = jnp.pad(x.reshape(M, 3), ((0, 0), (0, CP - 3)))

    x1p = _edge_layer(xpad, 3, W1, g1, b1, CP, pat)
    x2p = _edge_layer(x1p, 64, W2, g2, b2, CP, pat)
    x3p = _edge_layer(x2p, 64, W3, g3, b3, CP, pat)
    x4 = _edge_layer(x3p, 128, W4, g4, b4, 256, pat)

    xcat = jnp.concatenate(
        [x1p[:, :64], x2p[:, :64], x3p, x4], axis=1)
    y5, mean5, sqv5 = _conv5_p1(xcat, W5.T)
    emb = W5.shape[0]
    hmax, hmean = _conv5_p2(y5.reshape(B, N, emb), mean5, sqv5,
                            g5.reshape(1, emb), b5.reshape(1, emb))
    h = jnp.concatenate([hmax.reshape(B, emb), hmean.reshape(B, emb)],
                        axis=1)
    return _head(h, L1.T, gl1, bl1, L2.T, gl2, bl2, L3.T, bL3)
